# baseline (speedup 1.0000x reference)
import functools

import jax
import jax.numpy as jnp
from jax import lax
from jax.experimental import pallas as pl
from jax.experimental.pallas import tpu as pltpu
from jax.experimental.pallas import tpu_sc as plsc

D = 768
H = 4
SQRT_K = 4
K = SQRT_K * SQRT_K
TOP_K = 2
N_TOK = 32768
NEG_INF = float("-inf")

TB = 2048

try:
    _SC_INFO = plsc.get_sparse_core_info()
    _NC, _NS = _SC_INFO.num_cores, _SC_INFO.num_subcores
except Exception:
    _NC, _NS = 2, 16
_NW = _NC * _NS
_CHUNK = N_TOK // _NW
_GROUPS = _CHUNK // 16


def _tc_body(x_ref, w_ref, m8_ref, scores_ref):
    x = x_ref[...]
    w = w_ref[...]
    s = lax.dot_general(
        x, w, (((1,), (0,)), ((), ())), preferred_element_type=jnp.float32
    )
    st = s.T

    m1 = (st[0:4] + st[4:8] + st[8:12] + st[12:16]) * (1.0 / H)
    m2 = (st[16:20] + st[20:24] + st[24:28] + st[28:32]) * (1.0 / H)

    scores_t = jnp.concatenate(
        [m1[0:1] + m2, m1[1:2] + m2, m1[2:3] + m2, m1[3:4] + m2], axis=0
    )
    scores_ref[...] = scores_t.T
    m8_ref[...] = jnp.concatenate([m1, m2], axis=0)


def _top2_lanes(a0, a1, a2, a3):
    zero = jnp.zeros((16,), jnp.int32)
    v1, i1 = a0, zero
    v2 = jnp.full((16,), NEG_INF, jnp.float32)
    i2 = zero
    for k, ak in ((1, a1), (2, a2), (3, a3)):
        kk = jnp.full((16,), k, jnp.int32)
        gt1 = ak > v1
        gt2 = ak > v2
        i2 = jnp.where(gt1, i1, jnp.where(gt2, kk, i2))
        v2 = jnp.where(gt1, v1, jnp.where(gt2, ak, v2))
        i1 = jnp.where(gt1, kk, i1)
        v1 = jnp.where(gt1, ak, v1)
    return v1, i1, v2, i2


def _sc_body(m8_hbm, idx_hbm, gates_hbm, mv, idxb, gateb):
    wid = lax.axis_index("s") * _NC + lax.axis_index("c")
    base = wid * _CHUNK
    pltpu.sync_copy(m8_hbm.at[:, pl.ds(base, _CHUNK)], mv)

    lanes = lax.iota(jnp.int32, 16)

    def group(g, _):
        off = g * 16
        a0 = mv[0, pl.ds(off, 16)]
        a1 = mv[1, pl.ds(off, 16)]
        a2 = mv[2, pl.ds(off, 16)]
        a3 = mv[3, pl.ds(off, 16)]
        b0 = mv[4, pl.ds(off, 16)]
        b1 = mv[5, pl.ds(off, 16)]
        b2 = mv[6, pl.ds(off, 16)]
        b3 = mv[7, pl.ds(off, 16)]

        v1a, i1a, v2a, i2a = _top2_lanes(a0, a1, a2, a3)
        v1b, i1b, v2b, i2b = _top2_lanes(b0, b1, b2, b3)

        top1_v = v1a + v1b
        top1_i = SQRT_K * i1a + i1b
        ca_v = v2a + v1b
        ca_i = SQRT_K * i2a + i1b
        cb_v = v1a + v2b
        cb_i = SQRT_K * i1a + i2b
        take_b = (cb_v > ca_v) | ((cb_v == ca_v) & (cb_i < ca_i))
        top2_v = jnp.where(take_b, cb_v, ca_v)
        top2_i = jnp.where(take_b, cb_i, ca_i)

        e = jnp.exp(top2_v - top1_v)
        g1 = 1.0 / (1.0 + e)
        g2 = e / (1.0 + e)

        pos = g * 32 + 2 * lanes
        plsc.store_scatter(idxb, [pos], top1_i)
        plsc.store_scatter(idxb, [pos + 1], top2_i)
        plsc.store_scatter(gateb, [pos], g1)
        plsc.store_scatter(gateb, [pos + 1], g2)
        return ()

    lax.fori_loop(0, _GROUPS, group, (), unroll=4)

    pltpu.sync_copy(idxb, idx_hbm.at[pl.ds(2 * base, 2 * _CHUNK)])
    pltpu.sync_copy(gateb, gates_hbm.at[pl.ds(2 * base, 2 * _CHUNK)])


def kernel(x, W1, W2):
    n_tok = x.shape[0]
    w_cat_t = jnp.concatenate([W1, W2], axis=0).T
    grid = (n_tok // TB,)
    m8, scores = pl.pallas_call(
        _tc_body,
        grid=grid,
        in_specs=[
            pl.BlockSpec((TB, D), lambda i: (i, 0)),
            pl.BlockSpec((D, 2 * H * SQRT_K), lambda i: (0, 0)),
        ],
        out_specs=[
            pl.BlockSpec((8, TB), lambda i: (0, i)),
            pl.BlockSpec((TB, K), lambda i: (i, 0)),
        ],
        out_shape=[
            jax.ShapeDtypeStruct((8, n_tok), jnp.float32),
            jax.ShapeDtypeStruct((n_tok, K), jnp.float32),
        ],
        compiler_params=pltpu.CompilerParams(
            dimension_semantics=("arbitrary",),
        ),
    )(x, w_cat_t)

    mesh = plsc.VectorSubcoreMesh(core_axis_name="c", subcore_axis_name="s")
    sc = functools.partial(
        pl.kernel,
        mesh=mesh,
        out_type=[
            jax.ShapeDtypeStruct((2 * n_tok,), jnp.int32),
            jax.ShapeDtypeStruct((2 * n_tok,), jnp.float32),
        ],
        scratch_types=[
            pltpu.VMEM((8, _CHUNK), jnp.float32),
            pltpu.VMEM((2 * _CHUNK,), jnp.int32),
            pltpu.VMEM((2 * _CHUNK,), jnp.float32),
        ],
        compiler_params=pltpu.CompilerParams(needs_layout_passes=False),
    )(_sc_body)
    idx_flat, gates_flat = sc(m8)
    topk_idx = idx_flat.reshape(n_tok, TOP_K)
    gates = gates_flat.reshape(n_tok, TOP_K)
    return (topk_idx, gates, scores)


# speedup vs baseline: 2.2241x; 2.2241x over previous
import jax
import jax.numpy as jnp
from jax import lax
from jax.experimental import pallas as pl
from jax.experimental.pallas import tpu as pltpu

D = 768
H = 4
SQRT_K = 4
K = SQRT_K * SQRT_K
TOP_K = 2
NEG_INF = float("-inf")

TB = 4096


def _body(x_ref, w_ref, idx_ref, gates_ref, scores_ref):
    x = x_ref[...]
    w = w_ref[...]
    s = lax.dot_general(
        x, w, (((1,), (0,)), ((), ())), preferred_element_type=jnp.float32
    )
    st = s.T

    m1 = (st[0:4] + st[4:8] + st[8:12] + st[12:16]) * (1.0 / H)
    m2 = (st[16:20] + st[20:24] + st[24:28] + st[28:32]) * (1.0 / H)

    scores_t = jnp.concatenate(
        [m1[0:1] + m2, m1[1:2] + m2, m1[2:3] + m2, m1[3:4] + m2], axis=0
    )
    scores_ref[...] = scores_t.T

    iota = lax.broadcasted_iota(jnp.int32, (SQRT_K, TB), 0)

    v1a = jnp.max(m1, axis=0, keepdims=True)
    i1a = jnp.min(jnp.where(m1 == v1a, iota, SQRT_K), axis=0, keepdims=True)
    m1m = jnp.where(iota == i1a, NEG_INF, m1)
    v2a = jnp.max(m1m, axis=0, keepdims=True)
    i2a = jnp.min(jnp.where(m1m == v2a, iota, SQRT_K), axis=0, keepdims=True)

    v1b = jnp.max(m2, axis=0, keepdims=True)
    i1b = jnp.min(jnp.where(m2 == v1b, iota, SQRT_K), axis=0, keepdims=True)
    m2m = jnp.where(iota == i1b, NEG_INF, m2)
    v2b = jnp.max(m2m, axis=0, keepdims=True)
    i2b = jnp.min(jnp.where(m2m == v2b, iota, SQRT_K), axis=0, keepdims=True)

    top1_v = v1a + v1b
    top1_i = SQRT_K * i1a + i1b
    ca_v = v2a + v1b
    ca_i = SQRT_K * i2a + i1b
    cb_v = v1a + v2b
    cb_i = SQRT_K * i1a + i2b
    take_b = (cb_v > ca_v) | ((cb_v == ca_v) & (cb_i < ca_i))
    top2_v = jnp.where(take_b, cb_v, ca_v)
    top2_i = jnp.where(take_b, cb_i, ca_i)

    e = jnp.exp(top2_v - top1_v)
    g1 = 1.0 / (1.0 + e)
    g2 = e / (1.0 + e)

    zi = jnp.zeros((8 - 2 * TOP_K + 2, TB), jnp.int32)
    zg = jnp.zeros((8 - 2 * TOP_K + 2, TB), jnp.float32)
    idx_ref[...] = jnp.concatenate([top1_i, top2_i, zi], axis=0)
    gates_ref[...] = jnp.concatenate([g1, g2, zg], axis=0)


def kernel(x, W1, W2):
    n_tok = x.shape[0]
    w_cat_t = jnp.concatenate([W1, W2], axis=0).T
    grid = (n_tok // TB,)
    idx8, gates8, scores = pl.pallas_call(
        _body,
        grid=grid,
        in_specs=[
            pl.BlockSpec((TB, D), lambda i: (i, 0)),
            pl.BlockSpec((D, 2 * H * SQRT_K), lambda i: (0, 0)),
        ],
        out_specs=[
            pl.BlockSpec((8, TB), lambda i: (0, i)),
            pl.BlockSpec((8, TB), lambda i: (0, i)),
            pl.BlockSpec((TB, K), lambda i: (i, 0)),
        ],
        out_shape=[
            jax.ShapeDtypeStruct((8, n_tok), jnp.int32),
            jax.ShapeDtypeStruct((8, n_tok), jnp.float32),
            jax.ShapeDtypeStruct((n_tok, K), jnp.float32),
        ],
        compiler_params=pltpu.CompilerParams(
            dimension_semantics=("arbitrary",),
        ),
    )(x, w_cat_t)
    topk_idx = idx8[:TOP_K].T
    gates = gates8[:TOP_K].T
    return (topk_idx, gates, scores)
